# baseline (speedup 1.0000x reference)
import jax
import jax.numpy as jnp
from jax import lax
from jax.experimental import pallas as pl
from jax.experimental.pallas import tpu as pltpu
from jax.experimental.pallas import tpu_sc as plsc

_NC = 2
_NS = 16
_NW = _NC * _NS
_NSLOT = 64
_ACC_ROWS = 72


def _sc_stage1(n_nodes, n_edges, epw,
               x_hbm, ei_hbm, ea_hbm, w1_hbm, b1_hbm, w2_hbm, b2_hbm,
               sn_hbm, canon_hbm,
               acc_hbm, xsl_hbm,
               ei_v, lut_v, hpos_v, hslot_v,
               w1_v, w2_v, b1_v, b2_v, sn_v, canon_v,
               sidx_v, didx_v, eidx_v, srows_v, drows_v, erows_v,
               acc_v, xrows_v, sem):
    wid = lax.axis_index("s") * _NC + lax.axis_index("c")
    ebase = wid * epw

    ewin = ((epw + 127) // 128 + 1) * 128
    astart = pl.multiple_of(
        jnp.minimum(ebase - ebase % 128, n_edges - ewin), 128)
    delta = ebase - astart
    pltpu.sync_copy(ei_hbm.at[:, pl.ds(astart, ewin)], ei_v)
    pltpu.sync_copy(w1_hbm, w1_v)
    pltpu.sync_copy(w2_hbm, w2_v)
    pltpu.sync_copy(b1_hbm, b1_v)
    pltpu.sync_copy(b2_hbm, b2_v)
    pltpu.sync_copy(sn_hbm, sn_v)
    pltpu.sync_copy(canon_hbm, canon_v)

    neg1 = jnp.full((16,), -1, jnp.int32)

    def lut_init(i, _):
        for u in range(5):
            lut_v[pl.ds((i * 5 + u) * 16, 16)] = neg1
        return 0
    lax.fori_loop(0, n_nodes // 80, lut_init, 0)
    for g in range(_NSLOT // 16):
        idx = sn_v[pl.ds(g * 16, 16)]
        val = canon_v[pl.ds(g * 16, 16)]
        plsc.store_scatter(lut_v, [idx], val)

    def acc_init(r, _):
        acc_v[r, pl.ds(0, 16)] = jnp.zeros((16,), jnp.float32)
        acc_v[r, pl.ds(16, 16)] = jnp.zeros((16,), jnp.float32)
        return 0
    lax.fori_loop(0, _ACC_ROWS, acc_init, 0)

    iota16 = lax.iota(jnp.int32, 16)

    def scan_body(g, cursor):
        for u in range(5):
            i = g * 5 + u
            s = ei_v[0, pl.ds(delta + i * 16, 16)]
            slot = plsc.load_gather(lut_v, [s])
            m = slot >= 0
            plsc.store_compressed(hpos_v.at[pl.ds(cursor, 16)],
                                  i * 16 + iota16, mask=m)
            plsc.store_compressed(hslot_v.at[pl.ds(cursor, 16)], slot,
                                  mask=m)
            cursor = cursor + plsc.all_reduce_population_count(m)[0]
        return cursor

    nhit = lax.fori_loop(0, epw // 80, scan_body, jnp.int32(0))

    plsc.store_scatter(hpos_v, [nhit + iota16], jnp.zeros((16,), jnp.int32))

    def batch_body(b, _):
        bbase = b * 16
        hp = hpos_v[pl.ds(bbase, 16)]
        zero16 = jnp.zeros((16,), jnp.int32)
        sidx_v[...] = plsc.load_gather(ei_v, [zero16, delta + hp])
        didx_v[...] = plsc.load_gather(ei_v, [zero16 + 1, delta + hp])
        eidx_v[...] = ebase + hp
        pltpu.make_async_copy(x_hbm.at[sidx_v], srows_v, sem).start()
        pltpu.make_async_copy(x_hbm.at[didx_v], drows_v, sem).start()
        pltpu.make_async_copy(ea_hbm.at[eidx_v], erows_v, sem).start()
        pltpu.make_async_copy(x_hbm.at[sidx_v], srows_v, sem).wait()
        pltpu.make_async_copy(x_hbm.at[didx_v], drows_v, sem).wait()
        pltpu.make_async_copy(ea_hbm.at[eidx_v], erows_v, sem).wait()
        rem = jnp.minimum(jnp.int32(16), nhit - bbase)

        def edge_body(j, _):
            h0 = b1_v[pl.ds(0, 16)]
            h1 = b1_v[pl.ds(16, 16)]
            sr = [srows_v[j, pl.ds(16 * t, 16)] for t in range(8)]
            dr = [drows_v[j, pl.ds(16 * t, 16)] for t in range(8)]
            er = erows_v[j, pl.ds(0, 16)]
            for k in range(128):
                a = sr[k // 16][k % 16]
                h0 = h0 + a * w1_v[k, pl.ds(0, 16)]
                h1 = h1 + a * w1_v[k, pl.ds(16, 16)]
            for k in range(128):
                a = dr[k // 16][k % 16]
                h0 = h0 + a * w1_v[128 + k, pl.ds(0, 16)]
                h1 = h1 + a * w1_v[128 + k, pl.ds(16, 16)]
            for k in range(16):
                a = er[k]
                h0 = h0 + a * w1_v[256 + k, pl.ds(0, 16)]
                h1 = h1 + a * w1_v[256 + k, pl.ds(16, 16)]
            h0 = jnp.maximum(h0, 0.0)
            h1 = jnp.maximum(h1, 0.0)
            m0 = b2_v[pl.ds(0, 16)]
            m1 = b2_v[pl.ds(16, 16)]
            for k in range(16):
                m0 = m0 + h0[k] * w2_v[k, pl.ds(0, 16)]
                m1 = m1 + h0[k] * w2_v[k, pl.ds(16, 16)]
            for k in range(16):
                m0 = m0 + h1[k] * w2_v[16 + k, pl.ds(0, 16)]
                m1 = m1 + h1[k] * w2_v[16 + k, pl.ds(16, 16)]
            slot = hslot_v[pl.ds(bbase + j, 16)][0]
            acc_v[slot, pl.ds(0, 16)] = acc_v[slot, pl.ds(0, 16)] + m0
            acc_v[slot, pl.ds(16, 16)] = acc_v[slot, pl.ds(16, 16)] + m1
            return 0

        lax.fori_loop(0, rem, edge_body, 0)
        return 0

    lax.fori_loop(0, (nhit + 15) // 16, batch_body, 0)

    pltpu.sync_copy(acc_v, acc_hbm.at[wid])

    @pl.when(wid == 0)
    def _():
        pltpu.make_async_copy(x_hbm.at[sn_v], xrows_v, sem).start()
        pltpu.make_async_copy(x_hbm.at[sn_v], xrows_v, sem).wait()
        pltpu.sync_copy(xrows_v, xsl_hbm)


def _tc_stage2(acc_ref, xsl_ref, oh_ref, wmu_ref, wsig_ref, wmu2_ref,
               wsig2_ref, bmu_ref, bsig_ref, bmu2_ref, bsig2_ref, high_ref,
               out_ref):
    accsum = jnp.sum(acc_ref[...], axis=0)[:_NSLOT]
    conv = jnp.dot(oh_ref[...], accsum,
                   preferred_element_type=jnp.float32)
    xsl = xsl_ref[...]
    conv24 = jnp.concatenate([conv[24:], conv[:24]], axis=0)
    xsl24 = jnp.concatenate([xsl[24:], xsl[:24]], axis=0)
    ef = jnp.concatenate([xsl, conv, xsl24, conv24], axis=1)
    z160 = jnp.zeros((160, 1), jnp.float32)
    w4 = jnp.concatenate(
        [wmu_ref[...], wsig_ref[...],
         jnp.concatenate([wmu2_ref[...], z160], axis=0),
         jnp.concatenate([wsig2_ref[...], z160], axis=0)], axis=1)
    pre = jnp.dot(ef, w4, preferred_element_type=jnp.float32)
    bmu = bmu_ref[0]
    bsig = bsig_ref[0]
    bmu2 = bmu2_ref[0]
    bsig2 = bsig2_ref[0]

    def softplus(z):
        return jnp.maximum(z, 0.0) + jnp.log1p(jnp.exp(-jnp.abs(z)))

    def dirich(a_pre, b_pre, ba, bb):
        alpha = softplus(a_pre + ba + 1e-20) + 1e-20
        beta = softplus(b_pre + bb + 1e-20) + 1e-20
        return alpha / (alpha + beta)

    dis = dirich(pre[:, 0:1], pre[:, 1:2], bmu, bsig)
    ordv = dirich(pre[:, 2:3], pre[:, 3:4], bmu2, bsig2)
    ord24 = jnp.concatenate([ordv[24:], ordv[:24]], axis=0)

    rid = lax.broadcasted_iota(jnp.int32, (_NSLOT, 1), 0)
    res = jnp.where(rid < 24, dis, ord24)
    eye = (lax.broadcasted_iota(jnp.int32, (_NSLOT, _NSLOT), 0)
           == lax.broadcasted_iota(jnp.int32, (_NSLOT, _NSLOT), 1)
           ).astype(jnp.float32)
    res_row = lax.dot_general(res, eye, (((0,), (0,)), ((), ())),
                              preferred_element_type=jnp.float32)
    out_ref[...] = res_row[:, :27] * high_ref[...]


def kernel(x, edge_index, edge_attr, W1, b1, W2, b2, Wmu, bmu, Wsig, bsig,
           Wmu2, bmu2, Wsig2, bsig2, edges, high):
    n_nodes = x.shape[0]
    n_edges = edge_index.shape[1]
    epw = n_edges // _NW

    tail = jnp.arange(n_nodes - 3, n_nodes, dtype=jnp.int32)
    sn = jnp.concatenate([edges[:, 0].astype(jnp.int32),
                          edges[:, 1].astype(jnp.int32), tail])
    sn64 = jnp.concatenate(
        [sn, jnp.broadcast_to(sn[0], (_NSLOT - sn.shape[0],))])
    eq = sn64[:, None] == sn64[None, :]
    canon = jnp.argmax(eq, axis=1).astype(jnp.int32)
    onehot = (canon[:, None] == jnp.arange(_NSLOT)[None, :]).astype(jnp.float32)

    f32 = jnp.float32
    mesh = plsc.VectorSubcoreMesh(core_axis_name="c", subcore_axis_name="s",
                                  num_cores=_NC, num_subcores=_NS)
    sc_fn = pl.kernel(
        lambda *refs: _sc_stage1(n_nodes, n_edges, epw, *refs),
        out_type=(jax.ShapeDtypeStruct((_NW, _ACC_ROWS, 32), f32),
                  jax.ShapeDtypeStruct((_NSLOT, 128), f32)),
        mesh=mesh,
        compiler_params=pltpu.CompilerParams(
            needs_layout_passes=False, use_tc_tiling_on_sc=False,
            allow_input_fusion=[False, False, True, False, False, False,
                                False, False, False]),
        scratch_types=[
            pltpu.VMEM((2, ((epw + 127) // 128 + 1) * 128), jnp.int32),
            pltpu.VMEM((n_nodes,), jnp.int32),
            pltpu.VMEM((epw + 16,), jnp.int32),
            pltpu.VMEM((epw + 16,), jnp.int32),
            pltpu.VMEM((272, 32), f32),
            pltpu.VMEM((32, 32), f32),
            pltpu.VMEM((32,), f32),
            pltpu.VMEM((32,), f32),
            pltpu.VMEM((_NSLOT,), jnp.int32),
            pltpu.VMEM((_NSLOT,), jnp.int32),
            pltpu.VMEM((16,), jnp.int32),
            pltpu.VMEM((16,), jnp.int32),
            pltpu.VMEM((16,), jnp.int32),
            pltpu.VMEM((16, 128), f32),
            pltpu.VMEM((16, 128), f32),
            pltpu.VMEM((16, 16), f32),
            pltpu.VMEM((_ACC_ROWS, 32), f32),
            pltpu.VMEM((_NSLOT, 128), f32),
            pltpu.SemaphoreType.DMA,
        ],
    )
    acc_all, xsl = sc_fn(x, edge_index, edge_attr,
                         W1, b1, W2, b2, sn64, canon)

    vspec = pl.BlockSpec(memory_space=pltpu.VMEM)
    sspec = pl.BlockSpec(memory_space=pltpu.SMEM)
    out = pl.pallas_call(
        _tc_stage2,
        out_shape=jax.ShapeDtypeStruct((1, 27), f32),
        in_specs=[vspec, vspec, vspec, vspec, vspec, vspec, vspec,
                  sspec, sspec, sspec, sspec, vspec],
    )(acc_all, xsl, onehot, Wmu, Wsig, Wmu2, Wsig2,
      bmu, bsig, bmu2, bsig2, high[None, :])

    return out


# speedup vs baseline: 1.0033x; 1.0033x over previous
import jax
import jax.numpy as jnp
from jax import lax
from jax.experimental import pallas as pl
from jax.experimental.pallas import tpu as pltpu
from jax.experimental.pallas import tpu_sc as plsc

_NC = 2
_NS = 16
_NW = _NC * _NS
_NSLOT = 64
_ACC_ROWS = 72


def _sc_stage1(n_nodes, n_edges, epw,
               x_hbm, ei_hbm, ea_hbm, w1_hbm, b1_hbm, w2_hbm, b2_hbm,
               sn_hbm, canon_hbm,
               acc_hbm, xsl_hbm,
               ei_v, lut_v, hpos_v, hslot_v,
               w1_v, w2_v, b1_v, b2_v, sn_v, canon_v,
               sidx_v, didx_v, eidx_v, srows_v, drows_v, erows_v,
               acc_v, xrows_v, sem):
    wid = lax.axis_index("s") * _NC + lax.axis_index("c")
    ebase = wid * epw

    ewin = ((epw + 127) // 128 + 1) * 128
    astart = pl.multiple_of(
        jnp.minimum(ebase - ebase % 128, n_edges - ewin), 128)
    delta = ebase - astart
    pltpu.sync_copy(ei_hbm.at[:, pl.ds(astart, ewin)], ei_v)
    pltpu.sync_copy(w1_hbm, w1_v)
    pltpu.sync_copy(w2_hbm, w2_v)
    pltpu.sync_copy(b1_hbm, b1_v)
    pltpu.sync_copy(b2_hbm, b2_v)
    pltpu.sync_copy(sn_hbm, sn_v)
    pltpu.sync_copy(canon_hbm, canon_v)

    neg1 = jnp.full((16,), -1, jnp.int32)

    def lut_init(i, _):
        for u in range(5):
            lut_v[pl.ds((i * 5 + u) * 16, 16)] = neg1
        return 0
    lax.fori_loop(0, n_nodes // 80, lut_init, 0)
    for g in range(_NSLOT // 16):
        idx = sn_v[pl.ds(g * 16, 16)]
        val = canon_v[pl.ds(g * 16, 16)]
        plsc.store_scatter(lut_v, [idx], val)

    def acc_init(r, _):
        acc_v[r, pl.ds(0, 16)] = jnp.zeros((16,), jnp.float32)
        acc_v[r, pl.ds(16, 16)] = jnp.zeros((16,), jnp.float32)
        return 0
    lax.fori_loop(0, _ACC_ROWS, acc_init, 0)

    iota16 = lax.iota(jnp.int32, 16)

    def scan_body(g, cursor):
        for u in range(5):
            i = g * 5 + u
            s = ei_v[0, pl.ds(delta + i * 16, 16)]
            slot = plsc.load_gather(lut_v, [s])
            m = slot >= 0
            plsc.store_compressed(hpos_v.at[pl.ds(cursor, 16)],
                                  i * 16 + iota16, mask=m)
            plsc.store_compressed(hslot_v.at[pl.ds(cursor, 16)], slot,
                                  mask=m)
            cursor = cursor + plsc.all_reduce_population_count(m)[0]
        return cursor

    nhit = lax.fori_loop(0, epw // 80, scan_body, jnp.int32(0))

    plsc.store_scatter(hpos_v, [nhit + iota16], jnp.zeros((16,), jnp.int32))

    def batch_body(b, _):
        bbase = b * 16
        hp = hpos_v[pl.ds(bbase, 16)]
        zero16 = jnp.zeros((16,), jnp.int32)
        sidx_v[...] = plsc.load_gather(ei_v, [zero16, delta + hp])
        didx_v[...] = plsc.load_gather(ei_v, [zero16 + 1, delta + hp])
        eidx_v[...] = ebase + hp
        pltpu.make_async_copy(x_hbm.at[sidx_v], srows_v, sem).start()
        pltpu.make_async_copy(x_hbm.at[didx_v], drows_v, sem).start()
        pltpu.make_async_copy(ea_hbm.at[eidx_v], erows_v, sem).start()
        pltpu.make_async_copy(x_hbm.at[sidx_v], srows_v, sem).wait()
        pltpu.make_async_copy(x_hbm.at[didx_v], drows_v, sem).wait()
        pltpu.make_async_copy(ea_hbm.at[eidx_v], erows_v, sem).wait()
        rem = jnp.minimum(jnp.int32(16), nhit - bbase)

        def edge_body(j, _):
            h0 = b1_v[pl.ds(0, 16)]
            h1 = b1_v[pl.ds(16, 16)]
            sr = [srows_v[j, pl.ds(16 * t, 16)] for t in range(8)]
            dr = [drows_v[j, pl.ds(16 * t, 16)] for t in range(8)]
            er = erows_v[j, pl.ds(0, 16)]
            for k in range(128):
                a = sr[k // 16][k % 16]
                h0 = h0 + a * w1_v[k, pl.ds(0, 16)]
                h1 = h1 + a * w1_v[k, pl.ds(16, 16)]
            for k in range(128):
                a = dr[k // 16][k % 16]
                h0 = h0 + a * w1_v[128 + k, pl.ds(0, 16)]
                h1 = h1 + a * w1_v[128 + k, pl.ds(16, 16)]
            for k in range(16):
                a = er[k]
                h0 = h0 + a * w1_v[256 + k, pl.ds(0, 16)]
                h1 = h1 + a * w1_v[256 + k, pl.ds(16, 16)]
            h0 = jnp.maximum(h0, 0.0)
            h1 = jnp.maximum(h1, 0.0)
            m0 = b2_v[pl.ds(0, 16)]
            m1 = b2_v[pl.ds(16, 16)]
            for k in range(16):
                m0 = m0 + h0[k] * w2_v[k, pl.ds(0, 16)]
                m1 = m1 + h0[k] * w2_v[k, pl.ds(16, 16)]
            for k in range(16):
                m0 = m0 + h1[k] * w2_v[16 + k, pl.ds(0, 16)]
                m1 = m1 + h1[k] * w2_v[16 + k, pl.ds(16, 16)]
            slot = hslot_v[pl.ds(bbase + j, 16)][0]
            acc_v[slot, pl.ds(0, 16)] = acc_v[slot, pl.ds(0, 16)] + m0
            acc_v[slot, pl.ds(16, 16)] = acc_v[slot, pl.ds(16, 16)] + m1
            return 0

        lax.fori_loop(0, rem, edge_body, 0)
        return 0

    lax.fori_loop(0, (nhit + 15) // 16, batch_body, 0)

    pltpu.sync_copy(acc_v, acc_hbm.at[wid])

    @pl.when(wid == 0)
    def _():
        pltpu.make_async_copy(x_hbm.at[sn_v], xrows_v, sem).start()
        pltpu.make_async_copy(x_hbm.at[sn_v], xrows_v, sem).wait()
        pltpu.sync_copy(xrows_v, xsl_hbm)


def _tc_stage2(acc_ref, xsl_ref, oh_ref, wmu_ref, wsig_ref, wmu2_ref,
               wsig2_ref, bmu_ref, bsig_ref, bmu2_ref, bsig2_ref, high_ref,
               out_ref):
    accsum = jnp.sum(acc_ref[...], axis=0)[:_NSLOT]
    conv = jnp.dot(oh_ref[...], accsum,
                   preferred_element_type=jnp.float32)
    xsl = xsl_ref[...]
    conv24 = jnp.concatenate([conv[24:], conv[:24]], axis=0)
    xsl24 = jnp.concatenate([xsl[24:], xsl[:24]], axis=0)
    ef = jnp.concatenate([xsl, conv, xsl24, conv24], axis=1)
    z160 = jnp.zeros((160, 1), jnp.float32)
    w4 = jnp.concatenate(
        [wmu_ref[...], wsig_ref[...],
         jnp.concatenate([wmu2_ref[...], z160], axis=0),
         jnp.concatenate([wsig2_ref[...], z160], axis=0)], axis=1)
    pre = jnp.dot(ef, w4, preferred_element_type=jnp.float32)
    bmu = bmu_ref[0]
    bsig = bsig_ref[0]
    bmu2 = bmu2_ref[0]
    bsig2 = bsig2_ref[0]

    def softplus(z):
        return jnp.maximum(z, 0.0) + jnp.log1p(jnp.exp(-jnp.abs(z)))

    def dirich(a_pre, b_pre, ba, bb):
        alpha = softplus(a_pre + ba + 1e-20) + 1e-20
        beta = softplus(b_pre + bb + 1e-20) + 1e-20
        return alpha / (alpha + beta)

    dis = dirich(pre[:, 0:1], pre[:, 1:2], bmu, bsig)
    ordv = dirich(pre[:, 2:3], pre[:, 3:4], bmu2, bsig2)
    ord24 = jnp.concatenate([ordv[24:], ordv[:24]], axis=0)

    rid = lax.broadcasted_iota(jnp.int32, (_NSLOT, 1), 0)
    res = jnp.where(rid < 24, dis, ord24)
    eye = (lax.broadcasted_iota(jnp.int32, (_NSLOT, _NSLOT), 0)
           == lax.broadcasted_iota(jnp.int32, (_NSLOT, _NSLOT), 1)
           ).astype(jnp.float32)
    res_row = lax.dot_general(res, eye, (((0,), (0,)), ((), ())),
                              preferred_element_type=jnp.float32)
    out_ref[...] = res_row[:, :27] * high_ref[...]


def kernel(x, edge_index, edge_attr, W1, b1, W2, b2, Wmu, bmu, Wsig, bsig,
           Wmu2, bmu2, Wsig2, bsig2, edges, high):
    n_nodes = x.shape[0]
    n_edges = edge_index.shape[1]
    epw = n_edges // _NW

    tail = jnp.arange(n_nodes - 3, n_nodes, dtype=jnp.int32)
    sn = jnp.concatenate([edges[:, 0].astype(jnp.int32),
                          edges[:, 1].astype(jnp.int32), tail])
    sn64 = jnp.concatenate(
        [sn, jnp.broadcast_to(sn[0], (_NSLOT - sn.shape[0],))])
    eq = sn64[:, None] == sn64[None, :]
    canon = jnp.argmax(eq, axis=1).astype(jnp.int32)
    onehot = (canon[:, None] == jnp.arange(_NSLOT)[None, :]).astype(jnp.float32)

    f32 = jnp.float32
    mesh = plsc.VectorSubcoreMesh(core_axis_name="c", subcore_axis_name="s",
                                  num_cores=_NC, num_subcores=_NS)
    sc_fn = pl.kernel(
        lambda *refs: _sc_stage1(n_nodes, n_edges, epw, *refs),
        out_type=(jax.ShapeDtypeStruct((_NW, _ACC_ROWS, 32), f32),
                  jax.ShapeDtypeStruct((_NSLOT, 128), f32)),
        mesh=mesh,
        compiler_params=pltpu.CompilerParams(needs_layout_passes=False,
                                             use_tc_tiling_on_sc=False),
        scratch_types=[
            pltpu.VMEM((2, ((epw + 127) // 128 + 1) * 128), jnp.int32),
            pltpu.VMEM((n_nodes,), jnp.int32),
            pltpu.VMEM((epw + 16,), jnp.int32),
            pltpu.VMEM((epw + 16,), jnp.int32),
            pltpu.VMEM((272, 32), f32),
            pltpu.VMEM((32, 32), f32),
            pltpu.VMEM((32,), f32),
            pltpu.VMEM((32,), f32),
            pltpu.VMEM((_NSLOT,), jnp.int32),
            pltpu.VMEM((_NSLOT,), jnp.int32),
            pltpu.VMEM((16,), jnp.int32),
            pltpu.VMEM((16,), jnp.int32),
            pltpu.VMEM((16,), jnp.int32),
            pltpu.VMEM((16, 128), f32),
            pltpu.VMEM((16, 128), f32),
            pltpu.VMEM((16, 16), f32),
            pltpu.VMEM((_ACC_ROWS, 32), f32),
            pltpu.VMEM((_NSLOT, 128), f32),
            pltpu.SemaphoreType.DMA,
        ],
    )
    acc_all, xsl = sc_fn(x, edge_index, edge_attr,
                         W1, b1, W2, b2, sn64, canon)

    vspec = pl.BlockSpec(memory_space=pltpu.VMEM)
    sspec = pl.BlockSpec(memory_space=pltpu.SMEM)
    out = pl.pallas_call(
        _tc_stage2,
        out_shape=jax.ShapeDtypeStruct((1, 27), f32),
        in_specs=[vspec, vspec, vspec, vspec, vspec, vspec, vspec,
                  sspec, sspec, sspec, sspec, vspec],
    )(acc_all, xsl, onehot, Wmu, Wsig, Wmu2, Wsig2,
      bmu, bsig, bmu2, bsig2, high[None, :])

    return out
